# Initial kernel scaffold; baseline (speedup 1.0000x reference)
import jax
import jax.numpy as jnp
from jax.experimental import pallas as pl


def kernel(x, gate_w, w1, w2, w3):
    raise NotImplementedError("write your pallas kernel here")


# speedup vs baseline: 1.4863x; 1.4863x over previous
import functools

import jax
import jax.numpy as jnp
from jax.experimental import pallas as pl
from jax.experimental.pallas import tpu as pltpu


def _sigmoid(z):
    return 1.0 / (1.0 + jnp.exp(-z))


def _dense_moe_body(x_ref, gw_ref, w1_ref, w3_ref, w2_ref, out_ref, *, tb):
    e = pl.program_id(0)
    t = pl.program_id(1)
    xb = x_ref[...]

    logits = jax.lax.dot_general(
        xb, gw_ref[...], (((1,), (1,)), ((), ())),
        preferred_element_type=jnp.float32)
    n_e = logits.shape[1]
    idx = jax.lax.broadcasted_iota(jnp.int32, logits.shape, 1)
    m1 = jnp.max(logits, axis=1, keepdims=True)
    am1 = jnp.min(jnp.where(logits == m1, idx, n_e), axis=1, keepdims=True)
    masked = jnp.where(idx == am1, -jnp.inf, logits)
    m2 = jnp.max(masked, axis=1, keepdims=True)
    am2 = jnp.min(jnp.where(masked == m2, idx, n_e), axis=1, keepdims=True)
    p1 = 1.0 / (1.0 + jnp.exp(m2 - m1))
    p2 = 1.0 - p1
    coef = (jnp.where(am1 == e, p1, 0.0) + jnp.where(am2 == e, p2, 0.0))

    xb16 = xb.astype(jnp.bfloat16)
    h1 = jax.lax.dot_general(
        xb16, w1_ref[0], (((1,), (1,)), ((), ())),
        preferred_element_type=jnp.float32)
    h3 = jax.lax.dot_general(
        xb16, w3_ref[0], (((1,), (1,)), ((), ())),
        preferred_element_type=jnp.float32)
    g = (h1 * _sigmoid(h1) * h3).astype(jnp.bfloat16)
    y = jax.lax.dot_general(
        g, w2_ref[0], (((1,), (1,)), ((), ())),
        preferred_element_type=jnp.float32)

    contrib = y * coef
    rows = pl.ds(t * tb, tb)

    @pl.when(e == 0)
    def _():
        out_ref[rows, :] = contrib

    @pl.when(e > 0)
    def _():
        out_ref[rows, :] += contrib


def kernel(x, gate_w, w1, w2, w3):
    b, t, d = x.shape
    n = b * t
    n_e, h, _ = w1.shape
    x_flat = x.reshape(n, d)
    w1b = w1.astype(jnp.bfloat16)
    w2b = w2.astype(jnp.bfloat16)
    w3b = w3.astype(jnp.bfloat16)

    tb = min(256, n)
    n_blocks = n // tb

    out = pl.pallas_call(
        functools.partial(_dense_moe_body, tb=tb),
        grid=(n_e, n_blocks),
        in_specs=[
            pl.BlockSpec((tb, d), lambda e, i: (i, 0)),
            pl.BlockSpec((n_e, d), lambda e, i: (0, 0)),
            pl.BlockSpec((1, h, d), lambda e, i: (e, 0, 0)),
            pl.BlockSpec((1, h, d), lambda e, i: (e, 0, 0)),
            pl.BlockSpec((1, d, h), lambda e, i: (e, 0, 0)),
        ],
        out_specs=pl.BlockSpec((n, d), lambda e, i: (0, 0)),
        out_shape=jax.ShapeDtypeStruct((n, d), jnp.float32),
        compiler_params=pltpu.CompilerParams(
            dimension_semantics=("arbitrary", "arbitrary"),
        ),
    )(x_flat, gate_w, w1b, w3b, w2b)
    return out.reshape(b, t, d)


# speedup vs baseline: 2.5165x; 1.6931x over previous
import functools

import jax
import jax.numpy as jnp
from jax import lax
from jax.experimental import pallas as pl
from jax.experimental.pallas import tpu as pltpu
from jax.experimental.pallas import tpu_sc as plsc

NUM_EXPERTS = 8
TOKEN_BLOCK = 256
CSUM_BLK = 128
SC_CORES = 2
SC_SUBCORES = 16
SC_TILES = SC_CORES * SC_SUBCORES
SC_CHUNK = 32


def _route_body(x_ref, gw_ref, pos_ref, coef_ref, bexp_ref, *, n, n_e, tb):
    xb = x_ref[...]
    logits = lax.dot_general(xb, gw_ref[...], (((1,), (1,)), ((), ())),
                             preferred_element_type=jnp.float32)
    idx = lax.broadcasted_iota(jnp.int32, logits.shape, 1)
    m1 = jnp.max(logits, axis=1, keepdims=True)
    am1 = jnp.min(jnp.where(logits == m1, idx, n_e), axis=1, keepdims=True)
    masked = jnp.where(idx == am1, -jnp.inf, logits)
    m2 = jnp.max(masked, axis=1, keepdims=True)
    am2 = jnp.min(jnp.where(masked == m2, idx, n_e), axis=1, keepdims=True)
    p1 = 1.0 / (1.0 + jnp.exp(m2 - m1))
    p2 = 1.0 - p1

    sel = jnp.logical_or(idx == am1, idx == am2).astype(jnp.float32)

    r_i = lax.broadcasted_iota(jnp.int32, (CSUM_BLK, CSUM_BLK), 0)
    c_i = lax.broadcasted_iota(jnp.int32, (CSUM_BLK, CSUM_BLK), 1)
    l_incl = (r_i >= c_i).astype(jnp.float32)
    pieces = []
    prefix = jnp.zeros((1, n_e), jnp.float32)
    for b in range(n // CSUM_BLK):
        c_b = lax.slice(sel, (b * CSUM_BLK, 0), ((b + 1) * CSUM_BLK, n_e))
        r_b = lax.dot_general(l_incl, c_b, (((1,), (0,)), ((), ())),
                              preferred_element_type=jnp.float32)
        pieces.append(r_b + prefix)
        prefix = prefix + lax.slice(r_b, (CSUM_BLK - 1, 0), (CSUM_BLK, n_e))
    rank_incl = jnp.concatenate(pieces, axis=0)
    rank_excl = rank_incl - sel

    cnt = prefix
    cnt_pad = jnp.floor((cnt + (tb - 1)) / tb) * tb
    e_r = lax.broadcasted_iota(jnp.int32, (n_e, n_e), 0)
    e_c = lax.broadcasted_iota(jnp.int32, (n_e, n_e), 1)
    strict = (e_r < e_c).astype(jnp.float32)
    start = lax.dot_general(cnt_pad, strict, (((1,), (0,)), ((), ())),
                            preferred_element_type=jnp.float32)

    posv = start + rank_excl
    pos0 = jnp.sum(jnp.where(idx == am1, posv, 0.0), axis=1, keepdims=True)
    pos1 = jnp.sum(jnp.where(idx == am2, posv, 0.0), axis=1, keepdims=True)
    pos_ref[...] = jnp.concatenate([pos0, pos1], axis=1).astype(jnp.int32)
    coef_ref[...] = jnp.concatenate([p1, p2], axis=1)

    eye = (e_r == e_c).astype(jnp.float32)
    start_col = lax.dot_general(eye, start, (((1,), (1,)), ((), ())),
                                preferred_element_type=jnp.float32)
    boff = lax.broadcasted_iota(jnp.int32, (1, 128), 1).astype(jnp.float32) * tb
    be = jnp.sum((start_col <= boff).astype(jnp.int32), axis=0,
                 keepdims=True) - 1
    bexp_ref[...] = be


def _route(x_flat, gate_w):
    n, d = x_flat.shape
    n_e = gate_w.shape[0]
    return pl.pallas_call(
        functools.partial(_route_body, n=n, n_e=n_e, tb=TOKEN_BLOCK),
        in_specs=[
            pl.BlockSpec((n, d), lambda: (0, 0)),
            pl.BlockSpec((n_e, d), lambda: (0, 0)),
        ],
        out_specs=[
            pl.BlockSpec((n, 2), lambda: (0, 0)),
            pl.BlockSpec((n, 2), lambda: (0, 0)),
            pl.BlockSpec((1, 128), lambda: (0, 0)),
        ],
        out_shape=[
            jax.ShapeDtypeStruct((n, 2), jnp.int32),
            jax.ShapeDtypeStruct((n, 2), jnp.float32),
            jax.ShapeDtypeStruct((1, 128), jnp.int32),
        ],
    )(x_flat, gate_w)


def _dispatch(x_flat, pos_t, coef_t, n_slots):
    n, d = x_flat.shape
    per_tile = n // SC_TILES
    n_chunks = per_tile // SC_CHUNK
    mesh = plsc.VectorSubcoreMesh(core_axis_name="c", subcore_axis_name="s")

    @functools.partial(
        pl.kernel, mesh=mesh,
        out_type=[
            jax.ShapeDtypeStruct((n_slots, d), jnp.float32),
            jax.ShapeDtypeStruct((n_slots,), jnp.float32),
        ],
        scratch_types=[
            pltpu.VMEM((2, SC_CHUNK), jnp.int32),
            pltpu.VMEM((2, SC_CHUNK), jnp.float32),
            pltpu.VMEM((SC_CHUNK, d), jnp.float32),
        ],
    )
    def dispatch(x_hbm, pos_hbm, coef_hbm, xs_hbm, scoef_hbm, idx_v, cf_v,
                 rows_v):
        wid = lax.axis_index("s") * SC_CORES + lax.axis_index("c")
        tile_base = wid * per_tile
        for c in range(n_chunks):
            base = tile_base + c * SC_CHUNK
            sl = pl.ds(base, SC_CHUNK)
            pltpu.sync_copy(pos_hbm.at[0, sl], idx_v.at[0])
            pltpu.sync_copy(pos_hbm.at[1, sl], idx_v.at[1])
            pltpu.sync_copy(coef_hbm.at[0, sl], cf_v.at[0])
            pltpu.sync_copy(coef_hbm.at[1, sl], cf_v.at[1])
            pltpu.sync_copy(x_hbm.at[sl], rows_v)
            pltpu.sync_copy(rows_v, xs_hbm.at[idx_v.at[0]])
            pltpu.sync_copy(rows_v, xs_hbm.at[idx_v.at[1]])
            pltpu.sync_copy(cf_v.at[0], scoef_hbm.at[idx_v.at[0]])
            pltpu.sync_copy(cf_v.at[1], scoef_hbm.at[idx_v.at[1]])

    return dispatch(x_flat, pos_t, coef_t)


def _sigmoid(z):
    return 1.0 / (1.0 + jnp.exp(-z))


def _ffn_body(bexp_ref, x_ref, cf_ref, w1_ref, w3_ref, w2_ref, y_ref):
    xb16 = x_ref[...].astype(jnp.bfloat16)
    h1 = lax.dot_general(xb16, w1_ref[0], (((1,), (1,)), ((), ())),
                         preferred_element_type=jnp.float32)
    h3 = lax.dot_general(xb16, w3_ref[0], (((1,), (1,)), ((), ())),
                         preferred_element_type=jnp.float32)
    g = (h1 * _sigmoid(h1) * h3).astype(jnp.bfloat16)
    y = lax.dot_general(g, w2_ref[0], (((1,), (1,)), ((), ())),
                        preferred_element_type=jnp.float32)
    y_ref[...] = y * cf_ref[0]


def _ffn(x_sorted, slot_coef3, w1b, w3b, w2b, bexp):
    n_slots, d = x_sorted.shape
    n_e, h, _ = w1b.shape
    tb = TOKEN_BLOCK
    nb = n_slots // tb
    grid_spec = pltpu.PrefetchScalarGridSpec(
        num_scalar_prefetch=1,
        grid=(nb,),
        in_specs=[
            pl.BlockSpec((tb, d), lambda i, be: (i, 0)),
            pl.BlockSpec((1, tb, 1), lambda i, be: (i, 0, 0)),
            pl.BlockSpec((1, h, d), lambda i, be: (be[i], 0, 0)),
            pl.BlockSpec((1, h, d), lambda i, be: (be[i], 0, 0)),
            pl.BlockSpec((1, d, h), lambda i, be: (be[i], 0, 0)),
        ],
        out_specs=pl.BlockSpec((tb, d), lambda i, be: (i, 0)),
    )
    return pl.pallas_call(
        _ffn_body,
        grid_spec=grid_spec,
        out_shape=jax.ShapeDtypeStruct((n_slots, d), jnp.float32),
        compiler_params=pltpu.CompilerParams(
            dimension_semantics=("arbitrary",),
        ),
    )(bexp, x_sorted, slot_coef3, w1b, w3b, w2b)


def _combine(yw, pos_t, n, d):
    per_tile = n // SC_TILES
    n_chunks = per_tile // SC_CHUNK
    lanes = d // 16
    mesh = plsc.VectorSubcoreMesh(core_axis_name="c", subcore_axis_name="s")

    @functools.partial(
        pl.kernel, mesh=mesh,
        out_type=jax.ShapeDtypeStruct((n, d), jnp.float32),
        scratch_types=[
            pltpu.VMEM((2, SC_CHUNK), jnp.int32),
            pltpu.VMEM((SC_CHUNK, d), jnp.float32),
            pltpu.VMEM((SC_CHUNK, d), jnp.float32),
        ],
    )
    def combine(yw_hbm, pos_hbm, out_hbm, idx_v, y0_v, y1_v):
        wid = lax.axis_index("s") * SC_CORES + lax.axis_index("c")
        tile_base = wid * per_tile
        for c in range(n_chunks):
            base = tile_base + c * SC_CHUNK
            sl = pl.ds(base, SC_CHUNK)
            pltpu.sync_copy(pos_hbm.at[0, sl], idx_v.at[0])
            pltpu.sync_copy(pos_hbm.at[1, sl], idx_v.at[1])
            pltpu.sync_copy(yw_hbm.at[idx_v.at[0]], y0_v)
            pltpu.sync_copy(yw_hbm.at[idx_v.at[1]], y1_v)

            def add_row(r, _):
                for j in range(lanes):
                    ls = pl.ds(j * 16, 16)
                    y0_v[r, ls] = y0_v[r, ls] + y1_v[r, ls]
                return _

            lax.fori_loop(0, SC_CHUNK, add_row, 0)
            pltpu.sync_copy(y0_v, out_hbm.at[sl])

    return combine(yw, pos_t)


def kernel(x, gate_w, w1, w2, w3):
    b, t, d = x.shape
    n = b * t
    n_e, h, _ = w1.shape
    tb = TOKEN_BLOCK
    n_slots = 2 * n + n_e * tb
    x_flat = x.reshape(n, d)

    pos, coef, bexp = _route(x_flat, gate_w)
    pos_t = pos.T
    coef_t = coef.T
    x_sorted, slot_coef = _dispatch(x_flat, pos_t, coef_t, n_slots)

    w1b = w1.astype(jnp.bfloat16)
    w2b = w2.astype(jnp.bfloat16)
    w3b = w3.astype(jnp.bfloat16)
    yw = _ffn(x_sorted, slot_coef.reshape(n_slots // tb, tb, 1),
              w1b, w3b, w2b, bexp.reshape(128))
    out = _combine(yw, pos_t, n, d)
    return out.reshape(b, t, d)
